# baseline (speedup 1.0000x reference)
import functools

import jax
import jax.numpy as jnp
from jax import lax
from jax.experimental import pallas as pl
from jax.experimental.pallas import tpu as pltpu
from jax.experimental.pallas import tpu_sc as plsc

_N = 8192
_B = 1024
_K8 = 8
_K4 = 4
_TILE = 256
_NTILES = _N // _TILE
_NC = 2
_NS = 16
_NW = _NC * _NS
_RPW = _N // _NW
_BPW = _B // _NW


def _topk_body(starts_ref, rows_ref, cols_ref, a_ref, rel_ref, n8_ref, r8_ref,
               w_ref, vals_ref, inds_ref, relv_ref):
    i = pl.program_id(0)
    r0 = i * _TILE
    w_ref[...] = jnp.abs(a_ref[...])

    def scatter_body(t, carry):
        r = rows_ref[t]
        c = cols_ref[t]
        rr = r - r0
        row = w_ref[pl.ds(rr, 1), :]
        col = lax.broadcasted_iota(jnp.int32, (1, _N), 1)
        w_ref[pl.ds(rr, 1), :] = jnp.where(col == c, 0.0, row)
        return carry

    lax.fori_loop(starts_ref[i], starts_ref[i + 1], scatter_body, 0)

    colio = lax.broadcasted_iota(jnp.int32, (_TILE, _N), 1)
    big = jnp.int32(1 << 30)
    idx = None
    for k in range(_K8):
        w = w_ref[...]
        if k:
            w = jnp.where(colio == idx, -1.0, w)
            w_ref[...] = w
        m = jnp.max(w, axis=1, keepdims=True)
        key = jnp.min(
            jnp.where(w == m, (colio << 6) | rel_ref[...], big),
            axis=1, keepdims=True)
        idx = key >> 6
        vals_ref[k, :] = m[:, 0]
        inds_ref[k, :] = idx[:, 0]
        relv_ref[k, :] = (key & 63)[:, 0]

    valid = vals_ref[...] > 0.0
    inds = inds_ref[...]
    relv = relv_ref[...]
    n8_ref[...] = jnp.where(valid, inds, inds[0:1, :])
    r8_ref[...] = jnp.maximum(
        jnp.where(valid, relv, relv[0:1, :]) - 1, 0)


def _run_topk(node_adj, rel_adj, starts, rows, cols):
    return pl.pallas_call(
        _topk_body,
        grid=(_NTILES,),
        in_specs=[
            pl.BlockSpec(memory_space=pltpu.SMEM),
            pl.BlockSpec(memory_space=pltpu.SMEM),
            pl.BlockSpec(memory_space=pltpu.SMEM),
            pl.BlockSpec((_TILE, _N), lambda i: (i, 0)),
            pl.BlockSpec((_TILE, _N), lambda i: (i, 0)),
        ],
        out_specs=[
            pl.BlockSpec((_K8, _TILE), lambda i: (0, i)),
            pl.BlockSpec((_K8, _TILE), lambda i: (0, i)),
        ],
        out_shape=[
            jax.ShapeDtypeStruct((_K8, _N), jnp.int32),
            jax.ShapeDtypeStruct((_K8, _N), jnp.int32),
        ],
        scratch_shapes=[
            pltpu.VMEM((_TILE, _N), jnp.float32),
            pltpu.VMEM((_K8, _TILE), jnp.float32),
            pltpu.VMEM((_K8, _TILE), jnp.int32),
            pltpu.VMEM((_K8, _TILE), jnp.int32),
        ],
    )(starts, rows, cols, node_adj, rel_adj)


def _chunked_gather(table_hbm, idx_v, dst_v, n, sem):
    copies = []
    for c in range(0, n, 128):
        w = min(128, n - c)
        copies.append(
            pltpu.async_copy(table_hbm.at[idx_v.at[pl.ds(c, w)]],
                             dst_v.at[pl.ds(c, w)], sem))
    for cp in copies:
        cp.wait()


def _hop_gathers(idx_hbm, n8_hbm, r8_hbm, o1_hbm, or1_hbm,
                 o2_hbm, or2_hbm, bi_v, f1_v, h1n_v, h1r_v, f2_v, h2n_v,
                 h2r_v, sem, b0, wid):
    pltpu.sync_copy(idx_hbm.at[pl.ds(b0, _BPW)], bi_v)

    for k in range(_K8):
        def f1_body(j, carry):
            sl = pl.ds(j * 16, 16)
            f1_v[pl.ds(k * _BPW + j * 16, 16)] = k * _N + bi_v[sl]
            return carry

        lax.fori_loop(0, _BPW // 16, f1_body, 0)
    _chunked_gather(n8_hbm, f1_v, h1n_v, _BPW * _K8, sem)
    _chunked_gather(r8_hbm, f1_v, h1r_v, _BPW * _K8, sem)
    pltpu.sync_copy(h1n_v, o1_hbm.at[pl.ds(wid * _BPW * _K8, _BPW * _K8)])
    pltpu.sync_copy(h1r_v, or1_hbm.at[pl.ds(wid * _BPW * _K8, _BPW * _K8)])

    n1 = _BPW * _K8
    for k4 in range(_K4):
        def f2_body(j, carry):
            f2_v[pl.ds(k4 * n1 + j * 16, 16)] = k4 * _N + h1n_v[pl.ds(j * 16,
                                                                      16)]
            return carry

        lax.fori_loop(0, n1 // 16, f2_body, 0)
    _chunked_gather(n8_hbm, f2_v, h2n_v, n1 * _K4, sem)
    _chunked_gather(r8_hbm, f2_v, h2r_v, n1 * _K4, sem)
    pltpu.sync_copy(h2n_v, o2_hbm.at[pl.ds(wid * n1 * _K4, n1 * _K4)])
    pltpu.sync_copy(h2r_v, or2_hbm.at[pl.ds(wid * n1 * _K4, n1 * _K4)])


def _stagec_body(m_hbm, d_hbm, n8_hbm, r8_hbm, m1_hbm,
                 mr1_hbm, m2_hbm, mr2_hbm, d1_hbm, dr1_hbm, d2_hbm, dr2_hbm,
                 bi_v, f1_v, h1n_v, h1r_v, f2_v, h2n_v, h2r_v, sem):
    wid = lax.axis_index("s") * _NC + lax.axis_index("c")
    b0 = wid * _BPW
    _hop_gathers(m_hbm, n8_hbm, r8_hbm, m1_hbm, mr1_hbm,
                 m2_hbm, mr2_hbm, bi_v, f1_v, h1n_v, h1r_v, f2_v, h2n_v,
                 h2r_v, sem, b0, wid)
    _hop_gathers(d_hbm, n8_hbm, r8_hbm, d1_hbm, dr1_hbm,
                 d2_hbm, dr2_hbm, bi_v, f1_v, h1n_v, h1r_v, f2_v, h2n_v,
                 h2r_v, sem, b0, wid)


def _run_stagec(m_node, d_node, n8, r8):
    mesh = plsc.VectorSubcoreMesh(core_axis_name="c", subcore_axis_name="s")
    fn = functools.partial(
        pl.kernel,
        out_type=[jax.ShapeDtypeStruct((_B * _K8,), jnp.int32),
                  jax.ShapeDtypeStruct((_B * _K8,), jnp.int32),
                  jax.ShapeDtypeStruct((_B * _K8 * _K4,), jnp.int32),
                  jax.ShapeDtypeStruct((_B * _K8 * _K4,), jnp.int32)] * 2,
        mesh=mesh,
        scratch_types=[
            pltpu.VMEM((_BPW,), jnp.int32),
            pltpu.VMEM((_BPW * _K8,), jnp.int32),
            pltpu.VMEM((_BPW * _K8,), jnp.int32),
            pltpu.VMEM((_BPW * _K8,), jnp.int32),
            pltpu.VMEM((_BPW * _K8 * _K4,), jnp.int32),
            pltpu.VMEM((_BPW * _K8 * _K4,), jnp.int32),
            pltpu.VMEM((_BPW * _K8 * _K4,), jnp.int32),
            pltpu.SemaphoreType.DMA,
        ],
    )(_stagec_body)
    return fn(m_node, d_node, n8, r8)


def kernel(m_node, d_node, node_adj, rel_adj):
    m_node = m_node.astype(jnp.int32)
    d_node = d_node.astype(jnp.int32)

    md = jnp.concatenate([m_node, d_node])
    dm = jnp.concatenate([d_node, m_node])
    order = jnp.argsort(md)
    rows = md[order]
    cols = dm[order]
    starts = jnp.searchsorted(
        rows, jnp.arange(_NTILES + 1, dtype=jnp.int32) * _TILE
    ).astype(jnp.int32)

    n8_t, r8_t = _run_topk(node_adj, rel_adj, starts, rows, cols)

    m1, mr1, m2, mr2, d1, dr1, d2, dr2 = _run_stagec(
        m_node, d_node, n8_t.reshape(-1), r8_t.reshape(-1))

    def _h1(x):
        return x.reshape(_NW, _K8, _BPW).transpose(0, 2, 1).reshape(_B, _K8)

    def _h2(x):
        return x.reshape(_NW, _K4, _K8, _BPW).transpose(0, 3, 2, 1).reshape(
            _B, _K8 * _K4)

    mnei = (m_node[:, None], _h1(m1), _h2(m2))
    mrel = (_h1(mr1), _h2(mr2))
    dnei = (d_node[:, None], _h1(d1), _h2(d2))
    drel = (_h1(dr1), _h2(dr2))
    return (mnei, mrel, dnei, drel)


# speedup vs baseline: 1.1083x; 1.1083x over previous
import functools

import jax
import jax.numpy as jnp
from jax import lax
from jax.experimental import pallas as pl
from jax.experimental.pallas import tpu as pltpu
from jax.experimental.pallas import tpu_sc as plsc

_N = 8192
_B = 1024
_K8 = 8
_K4 = 4
_TILE = 256
_NTILES = _N // _TILE
_NC = 2
_NS = 16
_NW = _NC * _NS
_RPW = _N // _NW
_BPW = _B // _NW


def _topk_body(starts_ref, rows_ref, cols_ref, a_ref, rel_ref, n8_ref, r8_ref,
               w_ref, p_ref, vals_ref, inds_ref, relv_ref):
    i = pl.program_id(0)
    r0 = i * _TILE
    w_ref[...] = jnp.abs(a_ref[...])
    colio = lax.broadcasted_iota(jnp.int32, (_TILE, _N), 1)
    p_ref[...] = (colio << 6) | rel_ref[...]

    def scatter_body(t, carry):
        r = rows_ref[t]
        c = cols_ref[t]
        rr = r - r0
        row = w_ref[pl.ds(rr, 1), :]
        col = lax.broadcasted_iota(jnp.int32, (1, _N), 1)
        w_ref[pl.ds(rr, 1), :] = jnp.where(col == c, 0.0, row)
        return carry

    lax.fori_loop(starts_ref[i], starts_ref[i + 1], scatter_body, 0)

    big = jnp.int32(1 << 30)
    idx = None
    for k in range(_K8):
        w = w_ref[...]
        if k:
            w = jnp.where(colio == idx, -1.0, w)
            w_ref[...] = w
        m = jnp.max(w, axis=1, keepdims=True)
        key = jnp.min(
            jnp.where(w == m, p_ref[...], big),
            axis=1, keepdims=True)
        idx = key >> 6
        vals_ref[k, :] = m[:, 0]
        inds_ref[k, :] = idx[:, 0]
        relv_ref[k, :] = (key & 63)[:, 0]

    valid = vals_ref[...] > 0.0
    inds = inds_ref[...]
    relv = relv_ref[...]
    n8_ref[...] = jnp.where(valid, inds, inds[0:1, :])
    r8_ref[...] = jnp.maximum(
        jnp.where(valid, relv, relv[0:1, :]) - 1, 0)


def _run_topk(node_adj, rel_adj, starts, rows, cols):
    return pl.pallas_call(
        _topk_body,
        grid=(_NTILES,),
        in_specs=[
            pl.BlockSpec(memory_space=pltpu.SMEM),
            pl.BlockSpec(memory_space=pltpu.SMEM),
            pl.BlockSpec(memory_space=pltpu.SMEM),
            pl.BlockSpec((_TILE, _N), lambda i: (i, 0)),
            pl.BlockSpec((_TILE, _N), lambda i: (i, 0)),
        ],
        out_specs=[
            pl.BlockSpec((_K8, _TILE), lambda i: (0, i)),
            pl.BlockSpec((_K8, _TILE), lambda i: (0, i)),
        ],
        out_shape=[
            jax.ShapeDtypeStruct((_K8, _N), jnp.int32),
            jax.ShapeDtypeStruct((_K8, _N), jnp.int32),
        ],
        scratch_shapes=[
            pltpu.VMEM((_TILE, _N), jnp.float32),
            pltpu.VMEM((_TILE, _N), jnp.int32),
            pltpu.VMEM((_K8, _TILE), jnp.float32),
            pltpu.VMEM((_K8, _TILE), jnp.int32),
            pltpu.VMEM((_K8, _TILE), jnp.int32),
        ],
    )(starts, rows, cols, node_adj, rel_adj)


def _chunked_gather(table_hbm, idx_v, dst_v, n, sem):
    copies = []
    for c in range(0, n, 128):
        w = min(128, n - c)
        copies.append(
            pltpu.async_copy(table_hbm.at[idx_v.at[pl.ds(c, w)]],
                             dst_v.at[pl.ds(c, w)], sem))
    for cp in copies:
        cp.wait()


def _hop_gathers(idx_hbm, n8_hbm, r8_hbm, o1_hbm, or1_hbm,
                 o2_hbm, or2_hbm, bi_v, f1_v, h1n_v, h1r_v, f2_v, h2n_v,
                 h2r_v, sem, b0, wid):
    pltpu.sync_copy(idx_hbm.at[pl.ds(b0, _BPW)], bi_v)

    for k in range(_K8):
        def f1_body(j, carry):
            sl = pl.ds(j * 16, 16)
            f1_v[pl.ds(k * _BPW + j * 16, 16)] = k * _N + bi_v[sl]
            return carry

        lax.fori_loop(0, _BPW // 16, f1_body, 0)
    _chunked_gather(n8_hbm, f1_v, h1n_v, _BPW * _K8, sem)
    _chunked_gather(r8_hbm, f1_v, h1r_v, _BPW * _K8, sem)
    pltpu.sync_copy(h1n_v, o1_hbm.at[pl.ds(wid * _BPW * _K8, _BPW * _K8)])
    pltpu.sync_copy(h1r_v, or1_hbm.at[pl.ds(wid * _BPW * _K8, _BPW * _K8)])

    n1 = _BPW * _K8
    for k4 in range(_K4):
        def f2_body(j, carry):
            f2_v[pl.ds(k4 * n1 + j * 16, 16)] = k4 * _N + h1n_v[pl.ds(j * 16,
                                                                      16)]
            return carry

        lax.fori_loop(0, n1 // 16, f2_body, 0)
    _chunked_gather(n8_hbm, f2_v, h2n_v, n1 * _K4, sem)
    _chunked_gather(r8_hbm, f2_v, h2r_v, n1 * _K4, sem)
    pltpu.sync_copy(h2n_v, o2_hbm.at[pl.ds(wid * n1 * _K4, n1 * _K4)])
    pltpu.sync_copy(h2r_v, or2_hbm.at[pl.ds(wid * n1 * _K4, n1 * _K4)])


def _stagec_body(m_hbm, d_hbm, n8_hbm, r8_hbm, m1_hbm,
                 mr1_hbm, m2_hbm, mr2_hbm, d1_hbm, dr1_hbm, d2_hbm, dr2_hbm,
                 bi_v, f1_v, h1n_v, h1r_v, f2_v, h2n_v, h2r_v, sem):
    wid = lax.axis_index("s") * _NC + lax.axis_index("c")
    b0 = wid * _BPW
    _hop_gathers(m_hbm, n8_hbm, r8_hbm, m1_hbm, mr1_hbm,
                 m2_hbm, mr2_hbm, bi_v, f1_v, h1n_v, h1r_v, f2_v, h2n_v,
                 h2r_v, sem, b0, wid)
    _hop_gathers(d_hbm, n8_hbm, r8_hbm, d1_hbm, dr1_hbm,
                 d2_hbm, dr2_hbm, bi_v, f1_v, h1n_v, h1r_v, f2_v, h2n_v,
                 h2r_v, sem, b0, wid)


def _run_stagec(m_node, d_node, n8, r8):
    mesh = plsc.VectorSubcoreMesh(core_axis_name="c", subcore_axis_name="s")
    fn = functools.partial(
        pl.kernel,
        out_type=[jax.ShapeDtypeStruct((_B * _K8,), jnp.int32),
                  jax.ShapeDtypeStruct((_B * _K8,), jnp.int32),
                  jax.ShapeDtypeStruct((_B * _K8 * _K4,), jnp.int32),
                  jax.ShapeDtypeStruct((_B * _K8 * _K4,), jnp.int32)] * 2,
        mesh=mesh,
        scratch_types=[
            pltpu.VMEM((_BPW,), jnp.int32),
            pltpu.VMEM((_BPW * _K8,), jnp.int32),
            pltpu.VMEM((_BPW * _K8,), jnp.int32),
            pltpu.VMEM((_BPW * _K8,), jnp.int32),
            pltpu.VMEM((_BPW * _K8 * _K4,), jnp.int32),
            pltpu.VMEM((_BPW * _K8 * _K4,), jnp.int32),
            pltpu.VMEM((_BPW * _K8 * _K4,), jnp.int32),
            pltpu.SemaphoreType.DMA,
        ],
    )(_stagec_body)
    return fn(m_node, d_node, n8, r8)


def kernel(m_node, d_node, node_adj, rel_adj):
    m_node = m_node.astype(jnp.int32)
    d_node = d_node.astype(jnp.int32)

    md = jnp.concatenate([m_node, d_node])
    dm = jnp.concatenate([d_node, m_node])
    order = jnp.argsort(md)
    rows = md[order]
    cols = dm[order]
    starts = jnp.searchsorted(
        rows, jnp.arange(_NTILES + 1, dtype=jnp.int32) * _TILE
    ).astype(jnp.int32)

    n8_t, r8_t = _run_topk(node_adj, rel_adj, starts, rows, cols)

    m1, mr1, m2, mr2, d1, dr1, d2, dr2 = _run_stagec(
        m_node, d_node, n8_t.reshape(-1), r8_t.reshape(-1))

    def _h1(x):
        return x.reshape(_NW, _K8, _BPW).transpose(0, 2, 1).reshape(_B, _K8)

    def _h2(x):
        return x.reshape(_NW, _K4, _K8, _BPW).transpose(0, 3, 2, 1).reshape(
            _B, _K8 * _K4)

    mnei = (m_node[:, None], _h1(m1), _h2(m2))
    mrel = (_h1(mr1), _h2(mr2))
    dnei = (d_node[:, None], _h1(d1), _h2(d2))
    drel = (_h1(dr1), _h2(dr2))
    return (mnei, mrel, dnei, drel)


# speedup vs baseline: 1.1137x; 1.0049x over previous
import functools

import jax
import jax.numpy as jnp
from jax import lax
from jax.experimental import pallas as pl
from jax.experimental.pallas import tpu as pltpu
from jax.experimental.pallas import tpu_sc as plsc

_N = 8192
_B = 1024
_K8 = 8
_K4 = 4
_TILE = 128
_NTILES = _N // _TILE
_NC = 2
_NS = 16
_NW = _NC * _NS
_RPW = _N // _NW
_BPW = _B // _NW


def _topk_body(starts_ref, rows_ref, cols_ref, a_ref, rel_ref, n8_ref, r8_ref,
               w_ref, p_ref, vals_ref, inds_ref, relv_ref):
    i = pl.program_id(0)
    r0 = i * _TILE
    w_ref[...] = jnp.abs(a_ref[...])
    colio = lax.broadcasted_iota(jnp.int32, (_TILE, _N), 1)
    p_ref[...] = (colio << 6) | rel_ref[...]

    def scatter_body(t, carry):
        r = rows_ref[t]
        c = cols_ref[t]
        rr = r - r0
        row = w_ref[pl.ds(rr, 1), :]
        col = lax.broadcasted_iota(jnp.int32, (1, _N), 1)
        w_ref[pl.ds(rr, 1), :] = jnp.where(col == c, 0.0, row)
        return carry

    lax.fori_loop(starts_ref[i], starts_ref[i + 1], scatter_body, 0)

    big = jnp.int32(1 << 30)
    idx = None
    for k in range(_K8):
        w = w_ref[...]
        if k:
            w = jnp.where(colio == idx, -1.0, w)
            w_ref[...] = w
        m = jnp.max(w, axis=1, keepdims=True)
        key = jnp.min(
            jnp.where(w == m, p_ref[...], big),
            axis=1, keepdims=True)
        idx = key >> 6
        vals_ref[k, :] = m[:, 0]
        inds_ref[k, :] = idx[:, 0]
        relv_ref[k, :] = (key & 63)[:, 0]

    valid = vals_ref[...] > 0.0
    inds = inds_ref[...]
    relv = relv_ref[...]
    n8_ref[...] = jnp.where(valid, inds, inds[0:1, :])
    r8_ref[...] = jnp.maximum(
        jnp.where(valid, relv, relv[0:1, :]) - 1, 0)


def _run_topk(node_adj, rel_adj, starts, rows, cols):
    return pl.pallas_call(
        _topk_body,
        grid=(_NTILES,),
        in_specs=[
            pl.BlockSpec(memory_space=pltpu.SMEM),
            pl.BlockSpec(memory_space=pltpu.SMEM),
            pl.BlockSpec(memory_space=pltpu.SMEM),
            pl.BlockSpec((_TILE, _N), lambda i: (i, 0)),
            pl.BlockSpec((_TILE, _N), lambda i: (i, 0)),
        ],
        out_specs=[
            pl.BlockSpec((_K8, _TILE), lambda i: (0, i)),
            pl.BlockSpec((_K8, _TILE), lambda i: (0, i)),
        ],
        out_shape=[
            jax.ShapeDtypeStruct((_K8, _N), jnp.int32),
            jax.ShapeDtypeStruct((_K8, _N), jnp.int32),
        ],
        scratch_shapes=[
            pltpu.VMEM((_TILE, _N), jnp.float32),
            pltpu.VMEM((_TILE, _N), jnp.int32),
            pltpu.VMEM((_K8, _TILE), jnp.float32),
            pltpu.VMEM((_K8, _TILE), jnp.int32),
            pltpu.VMEM((_K8, _TILE), jnp.int32),
        ],
    )(starts, rows, cols, node_adj, rel_adj)


def _chunked_gather(table_hbm, idx_v, dst_v, n, sem):
    copies = []
    for c in range(0, n, 128):
        w = min(128, n - c)
        copies.append(
            pltpu.async_copy(table_hbm.at[idx_v.at[pl.ds(c, w)]],
                             dst_v.at[pl.ds(c, w)], sem))
    for cp in copies:
        cp.wait()


def _hop_gathers(idx_hbm, n8_hbm, r8_hbm, o1_hbm, or1_hbm,
                 o2_hbm, or2_hbm, bi_v, f1_v, h1n_v, h1r_v, f2_v, h2n_v,
                 h2r_v, sem, b0, wid):
    pltpu.sync_copy(idx_hbm.at[pl.ds(b0, _BPW)], bi_v)

    for k in range(_K8):
        def f1_body(j, carry):
            sl = pl.ds(j * 16, 16)
            f1_v[pl.ds(k * _BPW + j * 16, 16)] = k * _N + bi_v[sl]
            return carry

        lax.fori_loop(0, _BPW // 16, f1_body, 0)
    _chunked_gather(n8_hbm, f1_v, h1n_v, _BPW * _K8, sem)
    _chunked_gather(r8_hbm, f1_v, h1r_v, _BPW * _K8, sem)
    pltpu.sync_copy(h1n_v, o1_hbm.at[pl.ds(wid * _BPW * _K8, _BPW * _K8)])
    pltpu.sync_copy(h1r_v, or1_hbm.at[pl.ds(wid * _BPW * _K8, _BPW * _K8)])

    n1 = _BPW * _K8
    for k4 in range(_K4):
        def f2_body(j, carry):
            f2_v[pl.ds(k4 * n1 + j * 16, 16)] = k4 * _N + h1n_v[pl.ds(j * 16,
                                                                      16)]
            return carry

        lax.fori_loop(0, n1 // 16, f2_body, 0)
    _chunked_gather(n8_hbm, f2_v, h2n_v, n1 * _K4, sem)
    _chunked_gather(r8_hbm, f2_v, h2r_v, n1 * _K4, sem)
    pltpu.sync_copy(h2n_v, o2_hbm.at[pl.ds(wid * n1 * _K4, n1 * _K4)])
    pltpu.sync_copy(h2r_v, or2_hbm.at[pl.ds(wid * n1 * _K4, n1 * _K4)])


def _stagec_body(m_hbm, d_hbm, n8_hbm, r8_hbm, m1_hbm,
                 mr1_hbm, m2_hbm, mr2_hbm, d1_hbm, dr1_hbm, d2_hbm, dr2_hbm,
                 bi_v, f1_v, h1n_v, h1r_v, f2_v, h2n_v, h2r_v, sem):
    wid = lax.axis_index("s") * _NC + lax.axis_index("c")
    b0 = wid * _BPW
    _hop_gathers(m_hbm, n8_hbm, r8_hbm, m1_hbm, mr1_hbm,
                 m2_hbm, mr2_hbm, bi_v, f1_v, h1n_v, h1r_v, f2_v, h2n_v,
                 h2r_v, sem, b0, wid)
    _hop_gathers(d_hbm, n8_hbm, r8_hbm, d1_hbm, dr1_hbm,
                 d2_hbm, dr2_hbm, bi_v, f1_v, h1n_v, h1r_v, f2_v, h2n_v,
                 h2r_v, sem, b0, wid)


def _run_stagec(m_node, d_node, n8, r8):
    mesh = plsc.VectorSubcoreMesh(core_axis_name="c", subcore_axis_name="s")
    fn = functools.partial(
        pl.kernel,
        out_type=[jax.ShapeDtypeStruct((_B * _K8,), jnp.int32),
                  jax.ShapeDtypeStruct((_B * _K8,), jnp.int32),
                  jax.ShapeDtypeStruct((_B * _K8 * _K4,), jnp.int32),
                  jax.ShapeDtypeStruct((_B * _K8 * _K4,), jnp.int32)] * 2,
        mesh=mesh,
        scratch_types=[
            pltpu.VMEM((_BPW,), jnp.int32),
            pltpu.VMEM((_BPW * _K8,), jnp.int32),
            pltpu.VMEM((_BPW * _K8,), jnp.int32),
            pltpu.VMEM((_BPW * _K8,), jnp.int32),
            pltpu.VMEM((_BPW * _K8 * _K4,), jnp.int32),
            pltpu.VMEM((_BPW * _K8 * _K4,), jnp.int32),
            pltpu.VMEM((_BPW * _K8 * _K4,), jnp.int32),
            pltpu.SemaphoreType.DMA,
        ],
    )(_stagec_body)
    return fn(m_node, d_node, n8, r8)


def kernel(m_node, d_node, node_adj, rel_adj):
    m_node = m_node.astype(jnp.int32)
    d_node = d_node.astype(jnp.int32)

    md = jnp.concatenate([m_node, d_node])
    dm = jnp.concatenate([d_node, m_node])
    order = jnp.argsort(md)
    rows = md[order]
    cols = dm[order]
    starts = jnp.searchsorted(
        rows, jnp.arange(_NTILES + 1, dtype=jnp.int32) * _TILE
    ).astype(jnp.int32)

    n8_t, r8_t = _run_topk(node_adj, rel_adj, starts, rows, cols)

    m1, mr1, m2, mr2, d1, dr1, d2, dr2 = _run_stagec(
        m_node, d_node, n8_t.reshape(-1), r8_t.reshape(-1))

    def _h1(x):
        return x.reshape(_NW, _K8, _BPW).transpose(0, 2, 1).reshape(_B, _K8)

    def _h2(x):
        return x.reshape(_NW, _K4, _K8, _BPW).transpose(0, 3, 2, 1).reshape(
            _B, _K8 * _K4)

    mnei = (m_node[:, None], _h1(m1), _h2(m2))
    mrel = (_h1(mr1), _h2(mr2))
    dnei = (d_node[:, None], _h1(d1), _h2(d2))
    drel = (_h1(dr1), _h2(dr2))
    return (mnei, mrel, dnei, drel)


# speedup vs baseline: 1.2085x; 1.0851x over previous
import functools

import jax
import jax.numpy as jnp
from jax import lax
from jax.experimental import pallas as pl
from jax.experimental.pallas import tpu as pltpu
from jax.experimental.pallas import tpu_sc as plsc

_N = 8192
_B = 1024
_K8 = 8
_K4 = 4
_TILE = 128
_NTILES = _N // _TILE
_NC = 2
_NS = 16
_NW = _NC * _NS
_RPW = _N // _NW
_BPW = _B // _NW


def _topk_body(starts_ref, rows_ref, cols_ref, a_ref, rel_ref, n8_ref, r8_ref,
               w_ref, p_ref, vals_ref, inds_ref, relv_ref):
    i = pl.program_id(0)
    r0 = i * _TILE
    w_ref[...] = jnp.abs(a_ref[...])
    colio = lax.broadcasted_iota(jnp.int32, (_TILE, _N), 1)
    p_ref[...] = (colio << 6) | rel_ref[...]

    def scatter_body(t, carry):
        r = rows_ref[t]
        c = cols_ref[t]
        rr = r - r0
        row = w_ref[pl.ds(rr, 1), :]
        col = lax.broadcasted_iota(jnp.int32, (1, _N), 1)
        w_ref[pl.ds(rr, 1), :] = jnp.where(col == c, 0.0, row)
        return carry

    lax.fori_loop(starts_ref[i], starts_ref[i + 1], scatter_body, 0)

    big = jnp.int32(1 << 30)
    for k in range(_K8):
        w = w_ref[...]
        m = jnp.max(w, axis=1, keepdims=True)
        key = jnp.min(
            jnp.where(w == m, p_ref[...], big),
            axis=1, keepdims=True)
        idx = key >> 6
        vals_ref[k, :] = m[:, 0]
        inds_ref[k, :] = idx[:, 0]
        relv_ref[k, :] = (key & 63)[:, 0]
        if k + 1 < _K8:
            w_ref[...] = jnp.where(colio == idx, -1.0, w)

    valid = vals_ref[...] > 0.0
    inds = inds_ref[...]
    relv = relv_ref[...]
    n8_ref[...] = jnp.where(valid, inds, inds[0:1, :])
    r8_ref[...] = jnp.maximum(
        jnp.where(valid, relv, relv[0:1, :]) - 1, 0)


def _run_topk(node_adj, rel_adj, starts, rows, cols):
    return pl.pallas_call(
        _topk_body,
        grid=(_NTILES,),
        in_specs=[
            pl.BlockSpec(memory_space=pltpu.SMEM),
            pl.BlockSpec(memory_space=pltpu.SMEM),
            pl.BlockSpec(memory_space=pltpu.SMEM),
            pl.BlockSpec((_TILE, _N), lambda i: (i, 0)),
            pl.BlockSpec((_TILE, _N), lambda i: (i, 0)),
        ],
        out_specs=[
            pl.BlockSpec((_K8, _TILE), lambda i: (0, i)),
            pl.BlockSpec((_K8, _TILE), lambda i: (0, i)),
        ],
        out_shape=[
            jax.ShapeDtypeStruct((_K8, _N), jnp.int32),
            jax.ShapeDtypeStruct((_K8, _N), jnp.int32),
        ],
        scratch_shapes=[
            pltpu.VMEM((_TILE, _N), jnp.float32),
            pltpu.VMEM((_TILE, _N), jnp.int32),
            pltpu.VMEM((_K8, _TILE), jnp.float32),
            pltpu.VMEM((_K8, _TILE), jnp.int32),
            pltpu.VMEM((_K8, _TILE), jnp.int32),
        ],
    )(starts, rows, cols, node_adj, rel_adj)


def _chunked_gather(table_hbm, idx_v, dst_v, n, sem):
    copies = []
    for c in range(0, n, 128):
        w = min(128, n - c)
        copies.append(
            pltpu.async_copy(table_hbm.at[idx_v.at[pl.ds(c, w)]],
                             dst_v.at[pl.ds(c, w)], sem))
    for cp in copies:
        cp.wait()


def _hop_gathers(idx_hbm, n8_hbm, r8_hbm, o1_hbm, or1_hbm,
                 o2_hbm, or2_hbm, bi_v, f1_v, h1n_v, h1r_v, f2_v, h2n_v,
                 h2r_v, sem, b0, wid):
    pltpu.sync_copy(idx_hbm.at[pl.ds(b0, _BPW)], bi_v)

    for k in range(_K8):
        def f1_body(j, carry):
            sl = pl.ds(j * 16, 16)
            f1_v[pl.ds(k * _BPW + j * 16, 16)] = k * _N + bi_v[sl]
            return carry

        lax.fori_loop(0, _BPW // 16, f1_body, 0)
    _chunked_gather(n8_hbm, f1_v, h1n_v, _BPW * _K8, sem)
    _chunked_gather(r8_hbm, f1_v, h1r_v, _BPW * _K8, sem)
    pltpu.sync_copy(h1n_v, o1_hbm.at[pl.ds(wid * _BPW * _K8, _BPW * _K8)])
    pltpu.sync_copy(h1r_v, or1_hbm.at[pl.ds(wid * _BPW * _K8, _BPW * _K8)])

    n1 = _BPW * _K8
    for k4 in range(_K4):
        def f2_body(j, carry):
            f2_v[pl.ds(k4 * n1 + j * 16, 16)] = k4 * _N + h1n_v[pl.ds(j * 16,
                                                                      16)]
            return carry

        lax.fori_loop(0, n1 // 16, f2_body, 0)
    _chunked_gather(n8_hbm, f2_v, h2n_v, n1 * _K4, sem)
    _chunked_gather(r8_hbm, f2_v, h2r_v, n1 * _K4, sem)
    pltpu.sync_copy(h2n_v, o2_hbm.at[pl.ds(wid * n1 * _K4, n1 * _K4)])
    pltpu.sync_copy(h2r_v, or2_hbm.at[pl.ds(wid * n1 * _K4, n1 * _K4)])


def _stagec_body(m_hbm, d_hbm, n8_hbm, r8_hbm, m1_hbm,
                 mr1_hbm, m2_hbm, mr2_hbm, d1_hbm, dr1_hbm, d2_hbm, dr2_hbm,
                 bi_v, f1_v, h1n_v, h1r_v, f2_v, h2n_v, h2r_v, sem):
    wid = lax.axis_index("s") * _NC + lax.axis_index("c")
    b0 = wid * _BPW
    _hop_gathers(m_hbm, n8_hbm, r8_hbm, m1_hbm, mr1_hbm,
                 m2_hbm, mr2_hbm, bi_v, f1_v, h1n_v, h1r_v, f2_v, h2n_v,
                 h2r_v, sem, b0, wid)
    _hop_gathers(d_hbm, n8_hbm, r8_hbm, d1_hbm, dr1_hbm,
                 d2_hbm, dr2_hbm, bi_v, f1_v, h1n_v, h1r_v, f2_v, h2n_v,
                 h2r_v, sem, b0, wid)


def _run_stagec(m_node, d_node, n8, r8):
    mesh = plsc.VectorSubcoreMesh(core_axis_name="c", subcore_axis_name="s")
    fn = functools.partial(
        pl.kernel,
        out_type=[jax.ShapeDtypeStruct((_B * _K8,), jnp.int32),
                  jax.ShapeDtypeStruct((_B * _K8,), jnp.int32),
                  jax.ShapeDtypeStruct((_B * _K8 * _K4,), jnp.int32),
                  jax.ShapeDtypeStruct((_B * _K8 * _K4,), jnp.int32)] * 2,
        mesh=mesh,
        scratch_types=[
            pltpu.VMEM((_BPW,), jnp.int32),
            pltpu.VMEM((_BPW * _K8,), jnp.int32),
            pltpu.VMEM((_BPW * _K8,), jnp.int32),
            pltpu.VMEM((_BPW * _K8,), jnp.int32),
            pltpu.VMEM((_BPW * _K8 * _K4,), jnp.int32),
            pltpu.VMEM((_BPW * _K8 * _K4,), jnp.int32),
            pltpu.VMEM((_BPW * _K8 * _K4,), jnp.int32),
            pltpu.SemaphoreType.DMA,
        ],
    )(_stagec_body)
    return fn(m_node, d_node, n8, r8)


def kernel(m_node, d_node, node_adj, rel_adj):
    m_node = m_node.astype(jnp.int32)
    d_node = d_node.astype(jnp.int32)

    md = jnp.concatenate([m_node, d_node])
    dm = jnp.concatenate([d_node, m_node])
    order = jnp.argsort(md)
    rows = md[order]
    cols = dm[order]
    starts = jnp.searchsorted(
        rows, jnp.arange(_NTILES + 1, dtype=jnp.int32) * _TILE
    ).astype(jnp.int32)

    n8_t, r8_t = _run_topk(node_adj, rel_adj, starts, rows, cols)

    m1, mr1, m2, mr2, d1, dr1, d2, dr2 = _run_stagec(
        m_node, d_node, n8_t.reshape(-1), r8_t.reshape(-1))

    def _h1(x):
        return x.reshape(_NW, _K8, _BPW).transpose(0, 2, 1).reshape(_B, _K8)

    def _h2(x):
        return x.reshape(_NW, _K4, _K8, _BPW).transpose(0, 3, 2, 1).reshape(
            _B, _K8 * _K4)

    mnei = (m_node[:, None], _h1(m1), _h2(m2))
    mrel = (_h1(mr1), _h2(mr2))
    dnei = (d_node[:, None], _h1(d1), _h2(d2))
    drel = (_h1(dr1), _h2(dr2))
    return (mnei, mrel, dnei, drel)
